# baseline (speedup 1.0000x reference)
import jax
import jax.numpy as jnp
from jax import lax
from jax.experimental import pallas as pl
from jax.experimental.pallas import tpu as pltpu
from jax.experimental.pallas import tpu_sc as plsc

N = 10000
E = 320000
D = 128
NC = 2
NS = 16
NW = NC * NS
CH = 128
EPT = E // NW
NCHK = 80
EPT_PAD = NCHK * CH
PKW = (NCHK // 2) * CH
NP = 10016
AP = 10240
RPT = AP // NS

_MESH = dict(core_axis_name="c", subcore_axis_name="s",
             num_cores=NC, num_subcores=NS)
_PARAMS = pltpu.CompilerParams(needs_layout_passes=False)


def _tc_matmul_body(x_ref, w_ref, o_ref):
    o_ref[...] = jnp.dot(x_ref[...], w_ref[...],
                         preferred_element_type=jnp.float32)


def _tc_matmul(x, w, interpret=False):
    n, d = x.shape
    k = w.shape[1]
    blk = 400
    grid = n // blk
    return pl.pallas_call(
        _tc_matmul_body,
        interpret=interpret,
        grid=(grid,),
        in_specs=[
            pl.BlockSpec((blk, d), lambda i: (i, 0)),
            pl.BlockSpec((d, k), lambda i: (0, 0)),
        ],
        out_specs=pl.BlockSpec((blk, k), lambda i: (i, 0)),
        out_shape=jax.ShapeDtypeStruct((n, k), jnp.float32),
    )(x, w)


def _sc_agg_body(ys, sdb, zrow, aggs,
                 sdv, rows, acc_sh, gsem, ssem):
    c = lax.axis_index("c")
    s = lax.axis_index("s")

    pltpu.sync_copy(zrow, acc_sh.at[pl.ds(s * RPT, RPT)])
    plsc.subcore_barrier()

    def chunk(j, carry):
        pltpu.sync_copy(sdb.at[c, s, j], sdv)
        pltpu.async_copy(ys.at[sdv.at[0]], rows, gsem).wait()
        pltpu.async_copy(rows, acc_sh.at[sdv.at[1]], ssem, add=True).wait()
        return carry

    lax.fori_loop(0, NCHK, chunk, 0, unroll=False)
    plsc.subcore_barrier()

    pltpu.sync_copy(acc_sh.at[pl.ds(s * RPT, RPT)],
                    aggs.at[pl.ds(c * AP + s * RPT, RPT)])


def _sc_agg(ys, sdb, zrow, interpret=False):
    kfn = pl.kernel(
        _sc_agg_body,
        out_type=[jax.ShapeDtypeStruct((NC * AP, D), jnp.float32)],
        mesh=plsc.VectorSubcoreMesh(**_MESH),
        scratch_types=[
            pltpu.VMEM((2, CH), jnp.int32),
            pltpu.VMEM((CH, D), jnp.float32),
            pltpu.VMEM_SHARED((AP, D), jnp.float32),
            pltpu.SemaphoreType.DMA, pltpu.SemaphoreType.DMA,
        ],
        compiler_params=_PARAMS,
        interpret=interpret,
    )
    return kfn(ys, sdb, zrow)[0]


def _sc_cnt_body(dstf, zcnt, cnt_out,
                 dst_v, cnt_v, cbuf, res, cnt_all):
    c = lax.axis_index("c")
    s = lax.axis_index("s")

    pltpu.sync_copy(dstf.at[c, s], dst_v)
    pltpu.sync_copy(zcnt, cnt_v)
    ones = jnp.full((16,), 1.0, jnp.float32)

    def step(i, carry):
        d16 = dst_v[pl.ds(i * 16, 16)]
        plsc.addupdate_scatter(cnt_v, [d16], ones)
        return carry

    lax.fori_loop(0, EPT_PAD // 16, step, 0, unroll=False)

    for t in range(NS):
        pltpu.sync_copy(cnt_v.at[pl.ds(t * RPT, RPT)], cnt_all.at[t, s])
    plsc.subcore_barrier()

    pltpu.sync_copy(cnt_all.at[s], cbuf)

    def rbody(v, carry):
        acc = cbuf[0, pl.ds(v * 16, 16)]
        for t in range(1, NS):
            acc = acc + cbuf[t, pl.ds(v * 16, 16)]
        res[pl.ds(v * 16, 16)] = acc
        return carry

    lax.fori_loop(0, RPT // 16, rbody, 0, unroll=False)
    pltpu.sync_copy(res, cnt_out.at[pl.ds(c * AP + s * RPT, RPT)])


def _sc_cnt(dstf, zcnt, interpret=False):
    kfn = pl.kernel(
        _sc_cnt_body,
        out_type=[jax.ShapeDtypeStruct((NC * AP,), jnp.float32)],
        mesh=plsc.VectorSubcoreMesh(**_MESH),
        scratch_types=[
            pltpu.VMEM((EPT_PAD,), jnp.int32),
            pltpu.VMEM((AP,), jnp.float32),
            pltpu.VMEM((NS, RPT), jnp.float32),
            pltpu.VMEM((RPT,), jnp.float32),
            pltpu.VMEM_SHARED((NS, NS, RPT), jnp.float32),
        ],
        compiler_params=_PARAMS,
        interpret=interpret,
    )
    return kfn(dstf, zcnt)[0]


@jax.jit
def kernel(x, edge_index, Wl1, Wr1, b1, Wl2, Wr2, b2):
    src = edge_index[0]
    dst = edge_index[1]

    pad = jnp.full((NW, EPT_PAD - EPT), N, jnp.int32)
    src_t = jnp.concatenate([src.reshape(NW, EPT), pad], axis=1)
    dst_t = jnp.concatenate([dst.reshape(NW, EPT), pad], axis=1)
    sdb = jnp.stack([src_t.reshape(NW, NCHK, CH),
                     dst_t.reshape(NW, NCHK, CH)],
                    axis=2).reshape(NC, NS, NCHK, 2, CH)
    dstf = dst_t.reshape(NC, NS, EPT_PAD)
    zrow = jnp.zeros((RPT, D), jnp.float32)
    zcnt = jnp.zeros((AP,), jnp.float32)

    wcat1 = jnp.concatenate([Wl1.T, Wr1.T], axis=1)
    wcat2 = jnp.concatenate([Wl2.T, Wr2.T], axis=1)

    def padrows(xl):
        return jnp.pad(xl, ((0, NP - N), (0, 0)))

    cnt2 = _sc_cnt(dstf, zcnt)
    cnt = cnt2[:N] + cnt2[AP:AP + N]
    inv = 1.0 / jnp.maximum(cnt, 1.0)

    o1 = _tc_matmul(x, wcat1)
    xl1, z1 = o1[:, :D], o1[:, D:] + b1
    aggs1 = _sc_agg(padrows(xl1), sdb, zrow)
    agg1 = aggs1[:N] + aggs1[AP:AP + N]
    h = jax.nn.relu(agg1 * inv[:, None] + z1)

    o2 = _tc_matmul(h, wcat2)
    xl2, z2 = o2[:, :D], o2[:, D:] + b2
    aggs2 = _sc_agg(padrows(xl2), sdb, zrow)
    agg2 = aggs2[:N] + aggs2[AP:AP + N]
    return agg2 * inv[:, None] + z2


# speedup vs baseline: 1.2940x; 1.2940x over previous
import functools

import jax
import jax.numpy as jnp
from jax import lax
from jax.experimental import pallas as pl
from jax.experimental.pallas import tpu as pltpu
from jax.experimental.pallas import tpu_sc as plsc

N = 10000
E = 320000
D = 128
NC = 2
NS = 16
NW = NC * NS
CH = 128
EPT = E // NW
NCHK = (EPT + CH - 1) // CH
EPT_PAD = NCHK * CH
NP = 10016
AP = 10240
RPT = AP // NS


def _tc_matmul_body(x_ref, w_ref, o_ref):
    o_ref[...] = jnp.dot(x_ref[...], w_ref[...],
                         preferred_element_type=jnp.float32)


def _tc_matmul(x, w, interpret=False):
    n, d = x.shape
    k = w.shape[1]
    blk = 400
    grid = n // blk
    return pl.pallas_call(
        _tc_matmul_body,
        interpret=interpret,
        grid=(grid,),
        in_specs=[
            pl.BlockSpec((blk, d), lambda i: (i, 0)),
            pl.BlockSpec((d, k), lambda i: (0, 0)),
        ],
        out_specs=pl.BlockSpec((blk, k), lambda i: (i, 0)),
        out_shape=jax.ShapeDtypeStruct((n, k), jnp.float32),
    )(x, w)


def _sc_agg_body(with_cnt, *refs):
    if with_cnt:
        (ys, srcb, dstb, zrow, zcnt, aggs, cnt_out,
         idxg, idxs, rows_v, cnt_v, cbuf, res, acc_sh, cnt_all,
         gsem, ssem) = refs
    else:
        (ys, srcb, dstb, zrow, aggs,
         idxg, idxs, rows_v, acc_sh,
         gsem, ssem) = refs

    c = lax.axis_index("c")
    s = lax.axis_index("s")

    pltpu.sync_copy(zrow, acc_sh.at[pl.ds(s * RPT, RPT)])
    if with_cnt:
        pltpu.sync_copy(zcnt, cnt_v)
    plsc.subcore_barrier()

    ones = jnp.full((16,), 1.0, jnp.float32)

    def chunk(j, carry):
        pltpu.sync_copy(srcb.at[c, s, j], idxg)
        pltpu.sync_copy(dstb.at[c, s, j], idxs)
        pltpu.async_copy(ys.at[idxg], rows_v, gsem).wait()
        pltpu.async_copy(rows_v, acc_sh.at[idxs], ssem, add=True).wait()
        if with_cnt:
            for k in range(CH // 16):
                d16 = idxs[pl.ds(k * 16, 16)]
                plsc.addupdate_scatter(cnt_v, [d16], ones)
        return carry

    lax.fori_loop(0, NCHK, chunk, 0, unroll=False)

    if with_cnt:
        for t in range(NS):
            pltpu.sync_copy(cnt_v.at[pl.ds(t * RPT, RPT)], cnt_all.at[t, s])

    plsc.subcore_barrier()

    if with_cnt:
        pltpu.sync_copy(cnt_all.at[s], cbuf)

        def rbody(v, carry):
            acc = cbuf[0, pl.ds(v * 16, 16)]
            for t in range(1, NS):
                acc = acc + cbuf[t, pl.ds(v * 16, 16)]
            res[pl.ds(v * 16, 16)] = acc
            return carry

        lax.fori_loop(0, RPT // 16, rbody, 0, unroll=False)
        pltpu.sync_copy(res, cnt_out.at[pl.ds(c * AP + s * RPT, RPT)])

    pltpu.sync_copy(acc_sh.at[pl.ds(s * RPT, RPT)],
                    aggs.at[pl.ds(c * AP + s * RPT, RPT)])


def _sc_agg(ys, srcb, dstb, zrow, zcnt, with_cnt, interpret=False):
    mesh = plsc.VectorSubcoreMesh(core_axis_name="c", subcore_axis_name="s",
                                  num_cores=NC, num_subcores=NS)
    out_type = [jax.ShapeDtypeStruct((NC * AP, D), jnp.float32)]
    scratch = [
        pltpu.VMEM((CH,), jnp.int32),
        pltpu.VMEM((CH,), jnp.int32),
        pltpu.VMEM((CH, D), jnp.float32),
    ]
    if with_cnt:
        out_type.append(jax.ShapeDtypeStruct((NC * AP,), jnp.float32))
        scratch += [
            pltpu.VMEM((AP,), jnp.float32),
            pltpu.VMEM((NS, RPT), jnp.float32),
            pltpu.VMEM((RPT,), jnp.float32),
        ]
    scratch += [
        pltpu.VMEM_SHARED((AP, D), jnp.float32),
    ]
    if with_cnt:
        scratch.append(pltpu.VMEM_SHARED((NS, NS, RPT), jnp.float32))
    scratch += [pltpu.SemaphoreType.DMA, pltpu.SemaphoreType.DMA]

    if with_cnt:
        args = (ys, srcb, dstb, zrow, zcnt)
    else:
        args = (ys, srcb, dstb, zrow)

    kfn = pl.kernel(
        functools.partial(_sc_agg_body, with_cnt),
        out_type=out_type,
        mesh=mesh,
        scratch_types=scratch,
        compiler_params=pltpu.CompilerParams(needs_layout_passes=False),
        interpret=interpret,
    )
    return kfn(*args)


@jax.jit
def kernel(x, edge_index, Wl1, Wr1, b1, Wl2, Wr2, b2):
    src = edge_index[0]
    dst = edge_index[1]

    pad = jnp.full((NW, EPT_PAD - EPT), N, jnp.int32)
    srcb = jnp.concatenate([src.reshape(NW, EPT), pad],
                           axis=1).reshape(NC, NS, NCHK, CH)
    dstb = jnp.concatenate([dst.reshape(NW, EPT), pad],
                           axis=1).reshape(NC, NS, NCHK, CH)
    zrow = jnp.zeros((RPT, D), jnp.float32)
    zcnt = jnp.zeros((AP,), jnp.float32)

    wcat1 = jnp.concatenate([Wl1.T, Wr1.T], axis=1)
    wcat2 = jnp.concatenate([Wl2.T, Wr2.T], axis=1)

    def padrows(xl):
        return jnp.pad(xl, ((0, NP - N), (0, 0)))

    o1 = _tc_matmul(x, wcat1)
    xl1, z1 = o1[:, :D], o1[:, D:] + b1
    aggs1, cnt2 = _sc_agg(padrows(xl1), srcb, dstb, zrow, zcnt, True)
    agg1 = aggs1[:N] + aggs1[AP:AP + N]
    cnt = cnt2[:N] + cnt2[AP:AP + N]
    inv = 1.0 / jnp.maximum(cnt, 1.0)
    h = jax.nn.relu(agg1 * inv[:, None] + z1)

    o2 = _tc_matmul(h, wcat2)
    xl2, z2 = o2[:, :D], o2[:, D:] + b2
    aggs2 = _sc_agg(padrows(xl2), srcb, dstb, zrow, zcnt, False)[0]
    agg2 = aggs2[:N] + aggs2[AP:AP + N]
    return agg2 * inv[:, None] + z2
